# baseline (speedup 1.0000x reference)
import functools

import jax
import jax.numpy as jnp
from jax import lax
from jax.experimental import pallas as pl
from jax.experimental.pallas import tpu as pltpu
from jax.experimental.pallas import tpu_sc as plsc

N = 320000
D = 128
S = 10000

SB = 128
SBLOG = 7
CR = 2048
NB = (S + SB - 1) // SB
SPAD = NB * SB

NC = 2
NS = 16
NW = NC * NS
PT = N // NW
FP = 128

_NEG = float("-inf")


def _offs_scan_body(idx_hbm, fp_hbm, idxv, pvbuf, fp):
    wid = lax.axis_index("s") * NC + lax.axis_index("c")
    base = wid * PT
    pltpu.sync_copy(idx_hbm.at[pl.ds(pl.multiple_of(base, 8), PT)], idxv)
    pv_off = pl.multiple_of(jnp.maximum(base - 16, 0), 8)
    pltpu.sync_copy(idx_hbm.at[pl.ds(pv_off, 16)], pvbuf)
    for j in range(FP // 16):
        fp[pl.ds(j * 16, 16)] = jnp.full((16,), N, jnp.int32)
    prev0 = jnp.where(wid == 0, -1, pvbuf[...][15] >> SBLOG)
    ii = lax.iota(jnp.int32, 16)
    rotidx = (ii + 15) % 16

    def body(k, prev):
        v = idxv[pl.ds(k * 16, 16)]
        b = v >> SBLOG
        rot = b.at[rotidx].get(mode="promise_in_bounds")
        rot0 = jnp.where(ii == 0, prev, rot)
        mask = b != rot0
        pos = base + k * 16 + ii
        plsc.store_scatter(fp, [b], pos, mask=mask)
        return b[15]

    lax.fori_loop(0, PT // 16, body, prev0)
    pltpu.sync_copy(fp, fp_hbm.at[wid])


def _offs_merge_body(fp_hbm, offs_hbm, loc, res):
    wid = lax.axis_index("s") * NC + lax.axis_index("c")

    @pl.when(wid == 0)
    def _():
        pltpu.sync_copy(fp_hbm, loc)
        carry = jnp.int32(N)
        for j in range(FP // 16 - 1, -1, -1):
            m = loc[0, pl.ds(j * 16, 16)]
            for t in range(1, NW):
                m = jnp.minimum(m, loc[t, pl.ds(j * 16, 16)])
            cm = plsc.cummax(jnp.negative(lax.rev(m, (0,))))
            sm = lax.rev(jnp.negative(cm), (0,))
            s = jnp.minimum(sm, carry)
            res[pl.ds(j * 16, 16)] = s
            carry = s[0]
        pltpu.sync_copy(res, offs_hbm)


def _sc_offsets(idx1d):
    mesh = plsc.VectorSubcoreMesh(core_axis_name="c", subcore_axis_name="s")
    cp = pltpu.CompilerParams(needs_layout_passes=False)
    fp_all = pl.kernel(
        _offs_scan_body,
        out_type=jax.ShapeDtypeStruct((NW, FP), jnp.int32),
        mesh=mesh,
        compiler_params=cp,
        scratch_types=[
            pltpu.VMEM((PT,), jnp.int32),
            pltpu.VMEM((16,), jnp.int32),
            pltpu.VMEM((FP,), jnp.int32),
        ],
    )(idx1d)
    offs = pl.kernel(
        _offs_merge_body,
        out_type=jax.ShapeDtypeStruct((FP,), jnp.int32),
        mesh=mesh,
        compiler_params=cp,
        scratch_types=[
            pltpu.VMEM((NW, FP), jnp.int32),
            pltpu.VMEM((FP,), jnp.int32),
        ],
    )(fp_all)
    return offs[:NB + 1]


def _pool_body(offs_ref,
               idx_ref,
               x_ref,
               wg_ref,
               zn_ref,
               beta_ref,
               xbuf, ibuf, ssum, sem_x, sem_i):
    b = pl.program_id(0)
    start = offs_ref[b]
    end = offs_ref[b + 1]
    n = end - start

    zn_ref[...] = jnp.zeros((SB, D), jnp.float32)
    ssum[...] = jnp.zeros((SB, 1), jnp.float32)

    abase = (start >> 7) << 7
    nch = jax.lax.div(end - abase + CR - 1, CR)

    def chunk_off(c):
        return pl.multiple_of(jnp.minimum(abase + c * CR, N - CR), 128)

    def dma_pair(c, slot):
        o = chunk_off(c)
        return (
            pltpu.make_async_copy(x_ref.at[pl.ds(o, CR), :], xbuf.at[slot],
                                  sem_x.at[slot]),
            pltpu.make_async_copy(idx_ref.at[:, pl.ds(o, CR)], ibuf.at[slot],
                                  sem_i.at[slot]),
        )

    for k in range(3):
        @pl.when(k < nch)
        def _():
            for cp in dma_pair(k, k):
                cp.start()

    col = jax.lax.broadcasted_iota(jnp.int32, (1, CR), 1)

    def gate(c, c_prev):
        desired = abase + c * CR
        o = jnp.minimum(desired, N - CR)
        lo = jnp.maximum(start, desired) - o
        hi = jnp.minimum(end, desired + CR) - o
        xb = xbuf[jax.lax.rem(c, 4)]
        g = jnp.dot(xb, wg_ref[...], preferred_element_type=jnp.float32)
        gr = jnp.transpose(g)
        gr = jnp.where((col >= lo) & (col < hi), gr, _NEG)
        c_new = jnp.maximum(c_prev, jnp.max(gr))
        scale = jnp.exp(c_prev - c_new)
        e = jnp.exp(gr - c_new)
        return c_new, scale, e

    @pl.when(nch > 0)
    def _():
        for cp in dma_pair(0, 0):
            cp.wait()

    carry0 = gate(0, jnp.float32(_NEG))

    def body(c, carry):
        c_run, scale, e = carry
        slot = jax.lax.rem(c, 4)

        @pl.when(c + 3 < nch)
        def _():
            for cp in dma_pair(c + 3, jax.lax.rem(c + 3, 4)):
                cp.start()

        l = ibuf[slot] - b * SB
        oh = l == jax.lax.broadcasted_iota(jnp.int32, (SB, 1), 0)
        ohe = jnp.where(oh, e, 0.0).astype(jnp.bfloat16)
        rhs = jnp.concatenate(
            [xbuf[slot].astype(jnp.bfloat16), jnp.ones((CR, 1), jnp.bfloat16)],
            axis=1)
        acc = jnp.dot(ohe, rhs, preferred_element_type=jnp.float32)
        zn_ref[...] = zn_ref[...] * scale + acc[:, :D]
        ssum[...] = ssum[...] * scale + acc[:, D:]

        @pl.when(c + 1 < nch)
        def _():
            for cp in dma_pair(c + 1, jax.lax.rem(c + 1, 4)):
                cp.wait()

        c_nxt, scale_nxt, e_nxt = gate(c + 1, c_run)
        keep = c + 1 < nch
        return (jnp.where(keep, c_nxt, c_run),
                jnp.where(keep, scale_nxt, scale),
                jnp.where(keep, e_nxt, e))

    jax.lax.fori_loop(0, nch, body, carry0)

    s_col = ssum[...]
    beta_ref[...] = s_col / (s_col + 1e-10)
    zn_ref[...] = zn_ref[...] / (s_col + 1e-10)


def _msg_body(zn_ref, beta_ref, wm_ref, bm_ref, out_ref):
    out_ref[...] = (
        jnp.dot(zn_ref[...], wm_ref[...], preferred_element_type=jnp.float32)
        + beta_ref[...] * bm_ref[...])


@functools.partial(jax.jit, static_argnames=("interpret",))
def kernel(x, index, Wg, bg, Wm, bm, interpret=False):
    idx1d = index.astype(jnp.int32)
    idx = idx1d.reshape(1, N)
    if interpret:
        boundaries = jnp.arange(NB + 1, dtype=jnp.int32) * SB
        offs = jnp.searchsorted(idx1d, boundaries, side="left").astype(jnp.int32)
    else:
        offs = _sc_offsets(idx1d)

    grid_spec = pltpu.PrefetchScalarGridSpec(
        num_scalar_prefetch=1,
        grid=(NB,),
        in_specs=[
            pl.BlockSpec(memory_space=pl.ANY),
            pl.BlockSpec(memory_space=pl.ANY),
            pl.BlockSpec((D, 1), lambda b, offs: (0, 0)),
        ],
        out_specs=[
            pl.BlockSpec((SB, D), lambda b, offs: (b, 0)),
            pl.BlockSpec((SB, 1), lambda b, offs: (b, 0)),
        ],
        scratch_shapes=[
            pltpu.VMEM((4, CR, D), jnp.float32),
            pltpu.VMEM((4, 1, CR), jnp.int32),
            pltpu.VMEM((SB, 1), jnp.float32),
            pltpu.SemaphoreType.DMA((4,)),
            pltpu.SemaphoreType.DMA((4,)),
        ],
    )
    zn, beta = pl.pallas_call(
        _pool_body,
        grid_spec=grid_spec,
        out_shape=[
            jax.ShapeDtypeStruct((SPAD, D), jnp.float32),
            jax.ShapeDtypeStruct((SPAD, 1), jnp.float32),
        ],
        interpret=interpret,
    )(offs, idx, x, Wg)

    out = pl.pallas_call(
        _msg_body,
        out_shape=jax.ShapeDtypeStruct((SPAD, D), jnp.float32),
        interpret=interpret,
    )(zn, beta, Wm, bm.reshape(1, D))
    return out[:S]


# speedup vs baseline: 2.0255x; 2.0255x over previous
import functools

import jax
import jax.numpy as jnp
from jax import lax
from jax.experimental import pallas as pl
from jax.experimental.pallas import tpu as pltpu
from jax.experimental.pallas import tpu_sc as plsc

N = 320000
D = 128
S = 10000

SB = 128
SBLOG = 7
CR = 2176
NB = (S + SB - 1) // SB
SPAD = NB * SB

NC = 2
NS = 16
NW = NC * NS
PT = N // NW
FP = 128

_NEG = float("-inf")


def _offs_scan_body(idx_hbm, fp_hbm, idxv, pvbuf, fp):
    wid = lax.axis_index("s") * NC + lax.axis_index("c")
    base = wid * PT
    pltpu.sync_copy(idx_hbm.at[pl.ds(pl.multiple_of(base, 8), PT)], idxv)
    pv_off = pl.multiple_of(jnp.maximum(base - 16, 0), 8)
    pltpu.sync_copy(idx_hbm.at[pl.ds(pv_off, 16)], pvbuf)
    for j in range(FP // 16):
        fp[pl.ds(j * 16, 16)] = jnp.full((16,), N, jnp.int32)
    prev0 = jnp.where(wid == 0, -1, pvbuf[...][15] >> SBLOG)
    ii = lax.iota(jnp.int32, 16)
    rotidx = (ii + 15) % 16

    def body(k, prev):
        v = idxv[pl.ds(k * 16, 16)]
        b = v >> SBLOG
        rot = b.at[rotidx].get(mode="promise_in_bounds")
        rot0 = jnp.where(ii == 0, prev, rot)
        mask = b != rot0
        pos = base + k * 16 + ii
        plsc.store_scatter(fp, [b], pos, mask=mask)
        return b[15]

    lax.fori_loop(0, PT // 16, body, prev0)
    pltpu.sync_copy(fp, fp_hbm.at[wid])


def _offs_merge_body(fp_hbm, offs_hbm, loc, res):
    wid = lax.axis_index("s") * NC + lax.axis_index("c")

    @pl.when(wid == 0)
    def _():
        pltpu.sync_copy(fp_hbm, loc)
        carry = jnp.int32(N)
        for j in range(FP // 16 - 1, -1, -1):
            m = loc[0, pl.ds(j * 16, 16)]
            for t in range(1, NW):
                m = jnp.minimum(m, loc[t, pl.ds(j * 16, 16)])
            cm = plsc.cummax(jnp.negative(lax.rev(m, (0,))))
            sm = lax.rev(jnp.negative(cm), (0,))
            s = jnp.minimum(sm, carry)
            res[pl.ds(j * 16, 16)] = s
            carry = s[0]
        pltpu.sync_copy(res, offs_hbm)


def _sc_offsets(idx1d):
    mesh = plsc.VectorSubcoreMesh(core_axis_name="c", subcore_axis_name="s")
    cp = pltpu.CompilerParams(needs_layout_passes=False)
    fp_all = pl.kernel(
        _offs_scan_body,
        out_type=jax.ShapeDtypeStruct((NW, FP), jnp.int32),
        mesh=mesh,
        compiler_params=cp,
        scratch_types=[
            pltpu.VMEM((PT,), jnp.int32),
            pltpu.VMEM((16,), jnp.int32),
            pltpu.VMEM((FP,), jnp.int32),
        ],
    )(idx1d)
    offs = pl.kernel(
        _offs_merge_body,
        out_type=jax.ShapeDtypeStruct((FP,), jnp.int32),
        mesh=mesh,
        compiler_params=cp,
        scratch_types=[
            pltpu.VMEM((NW, FP), jnp.int32),
            pltpu.VMEM((FP,), jnp.int32),
        ],
    )(fp_all)
    return offs[:NB + 1]


def _pool_body(offs_ref,
               idx_ref,
               x_ref,
               wg_ref,
               zn_ref,
               beta_ref,
               xbuf, ibuf, ssum, sem_x, sem_i):
    b = pl.program_id(0)
    start = offs_ref[b]
    end = offs_ref[b + 1]
    n = end - start

    zn_ref[...] = jnp.zeros((SB, D), jnp.float32)
    ssum[...] = jnp.zeros((SB, 1), jnp.float32)

    abase = (start >> 7) << 7
    nch = jax.lax.div(end - abase + CR - 1, CR)
    bank = jax.lax.rem(b, 2) * 4

    def mk_dma(abase_, c, slot):
        o = pl.multiple_of(jnp.minimum(abase_ + c * CR, N - CR), 128)
        return (
            pltpu.make_async_copy(x_ref.at[pl.ds(o, CR), :], xbuf.at[slot],
                                  sem_x.at[slot]),
            pltpu.make_async_copy(idx_ref.at[:, pl.ds(o, CR)], ibuf.at[slot],
                                  sem_i.at[slot]),
        )

    @pl.when(b == 0)
    def _():
        for k in range(3):
            @pl.when(k < nch)
            def _():
                for cp in mk_dma(abase, k, k):
                    cp.start()

    @pl.when(b + 1 < NB)
    def _():
        s2 = offs_ref[b + 1]
        e2 = offs_ref[b + 2]
        ab2 = (s2 >> 7) << 7
        nch2 = jax.lax.div(e2 - ab2 + CR - 1, CR)
        obank = 4 - bank
        for k in range(3):
            @pl.when(k < nch2)
            def _():
                for cp in mk_dma(ab2, k, obank + k):
                    cp.start()

    col = jax.lax.broadcasted_iota(jnp.int32, (1, CR), 1)

    def body(c, c_old):
        slot = bank + jax.lax.rem(c, 4)
        desired = abase + c * CR
        o = jnp.minimum(desired, N - CR)

        @pl.when(c + 3 < nch)
        def _():
            for cp in mk_dma(abase, c + 3, bank + jax.lax.rem(c + 3, 4)):
                cp.start()

        for cp in mk_dma(abase, c, slot):
            cp.wait()

        lo = jnp.maximum(start, desired) - o
        hi = jnp.minimum(end, desired + CR) - o
        xb = xbuf[slot]
        g = jnp.dot(xb, wg_ref[...], preferred_element_type=jnp.float32)
        gr = jnp.transpose(g)
        gr = jnp.where((col >= lo) & (col < hi), gr, _NEG)
        c_new = jnp.maximum(c_old, jnp.max(gr))
        scale = jnp.exp(c_old - c_new)
        e = jnp.exp(gr - c_new)

        l = ibuf[slot] - b * SB
        oh = l == jax.lax.broadcasted_iota(jnp.int32, (SB, 1), 0)
        ohe = jnp.where(oh, e, 0.0).astype(jnp.bfloat16)
        rhs = jnp.concatenate(
            [xb.astype(jnp.bfloat16), jnp.ones((CR, 1), jnp.bfloat16)],
            axis=1)
        acc = jnp.dot(ohe, rhs, preferred_element_type=jnp.float32)
        zn_ref[...] = zn_ref[...] * scale + acc[:, :D]
        ssum[...] = ssum[...] * scale + acc[:, D:]
        return c_new

    jax.lax.fori_loop(0, nch, body, jnp.float32(_NEG))

    s_col = ssum[...]
    beta_ref[...] = s_col / (s_col + 1e-10)
    zn_ref[...] = zn_ref[...] / (s_col + 1e-10)


def _msg_body(zn_ref, beta_ref, wm_ref, bm_ref, out_ref):
    out_ref[...] = (
        jnp.dot(zn_ref[...], wm_ref[...], preferred_element_type=jnp.float32)
        + beta_ref[...] * bm_ref[...])


@functools.partial(jax.jit, static_argnames=("interpret",))
def kernel(x, index, Wg, bg, Wm, bm, interpret=False):
    idx1d = index.astype(jnp.int32)
    idx = idx1d.reshape(1, N)
    if interpret:
        boundaries = jnp.arange(NB + 1, dtype=jnp.int32) * SB
        offs = jnp.searchsorted(idx1d, boundaries, side="left").astype(jnp.int32)
    else:
        offs = _sc_offsets(idx1d)

    grid_spec = pltpu.PrefetchScalarGridSpec(
        num_scalar_prefetch=1,
        grid=(NB,),
        in_specs=[
            pl.BlockSpec(memory_space=pl.ANY),
            pl.BlockSpec(memory_space=pl.ANY),
            pl.BlockSpec((D, 1), lambda b, offs: (0, 0)),
        ],
        out_specs=[
            pl.BlockSpec((SB, D), lambda b, offs: (b, 0)),
            pl.BlockSpec((SB, 1), lambda b, offs: (b, 0)),
        ],
        scratch_shapes=[
            pltpu.VMEM((8, CR, D), jnp.float32),
            pltpu.VMEM((8, 1, CR), jnp.int32),
            pltpu.VMEM((SB, 1), jnp.float32),
            pltpu.SemaphoreType.DMA((8,)),
            pltpu.SemaphoreType.DMA((8,)),
        ],
    )
    zn, beta = pl.pallas_call(
        _pool_body,
        grid_spec=grid_spec,
        out_shape=[
            jax.ShapeDtypeStruct((SPAD, D), jnp.float32),
            jax.ShapeDtypeStruct((SPAD, 1), jnp.float32),
        ],
        interpret=interpret,
    )(offs, idx, x, Wg)

    out = pl.pallas_call(
        _msg_body,
        out_shape=jax.ShapeDtypeStruct((SPAD, D), jnp.float32),
        interpret=interpret,
    )(zn, beta, Wm, bm.reshape(1, D))
    return out[:S]


# speedup vs baseline: 2.4210x; 1.1953x over previous
import functools

import jax
import jax.numpy as jnp
from jax import lax
from jax.experimental import pallas as pl
from jax.experimental.pallas import tpu as pltpu
from jax.experimental.pallas import tpu_sc as plsc

N = 320000
D = 128
S = 10000

SB = 128
SBLOG = 7
CR = 4352
NB = (S + SB - 1) // SB
SPAD = NB * SB

NC = 2
NS = 16
NW = NC * NS
PT = N // NW
FP = 128

_NEG = float("-inf")


def _offs_scan_body(idx_hbm, fp_hbm, idxv, pvbuf, fp):
    wid = lax.axis_index("s") * NC + lax.axis_index("c")
    base = wid * PT
    pltpu.sync_copy(idx_hbm.at[pl.ds(pl.multiple_of(base, 8), PT)], idxv)
    pv_off = pl.multiple_of(jnp.maximum(base - 16, 0), 8)
    pltpu.sync_copy(idx_hbm.at[pl.ds(pv_off, 16)], pvbuf)
    for j in range(FP // 16):
        fp[pl.ds(j * 16, 16)] = jnp.full((16,), N, jnp.int32)
    prev0 = jnp.where(wid == 0, -1, pvbuf[...][15] >> SBLOG)
    ii = lax.iota(jnp.int32, 16)
    rotidx = (ii + 15) % 16

    def body(k, prev):
        v = idxv[pl.ds(k * 16, 16)]
        b = v >> SBLOG
        rot = b.at[rotidx].get(mode="promise_in_bounds")
        rot0 = jnp.where(ii == 0, prev, rot)
        mask = b != rot0
        pos = base + k * 16 + ii
        plsc.store_scatter(fp, [b], pos, mask=mask)
        return b[15]

    lax.fori_loop(0, PT // 16, body, prev0)
    pltpu.sync_copy(fp, fp_hbm.at[wid])


def _offs_merge_body(fp_hbm, offs_hbm, loc, res):
    wid = lax.axis_index("s") * NC + lax.axis_index("c")

    @pl.when(wid == 0)
    def _():
        pltpu.sync_copy(fp_hbm, loc)
        carry = jnp.int32(N)
        for j in range(FP // 16 - 1, -1, -1):
            m = loc[0, pl.ds(j * 16, 16)]
            for t in range(1, NW):
                m = jnp.minimum(m, loc[t, pl.ds(j * 16, 16)])
            cm = plsc.cummax(jnp.negative(lax.rev(m, (0,))))
            sm = lax.rev(jnp.negative(cm), (0,))
            s = jnp.minimum(sm, carry)
            res[pl.ds(j * 16, 16)] = s
            carry = s[0]
        pltpu.sync_copy(res, offs_hbm)


def _sc_offsets(idx1d):
    mesh = plsc.VectorSubcoreMesh(core_axis_name="c", subcore_axis_name="s")
    cp = pltpu.CompilerParams(needs_layout_passes=False)
    fp_all = pl.kernel(
        _offs_scan_body,
        out_type=jax.ShapeDtypeStruct((NW, FP), jnp.int32),
        mesh=mesh,
        compiler_params=cp,
        scratch_types=[
            pltpu.VMEM((PT,), jnp.int32),
            pltpu.VMEM((16,), jnp.int32),
            pltpu.VMEM((FP,), jnp.int32),
        ],
    )(idx1d)
    offs = pl.kernel(
        _offs_merge_body,
        out_type=jax.ShapeDtypeStruct((FP,), jnp.int32),
        mesh=mesh,
        compiler_params=cp,
        scratch_types=[
            pltpu.VMEM((NW, FP), jnp.int32),
            pltpu.VMEM((FP,), jnp.int32),
        ],
    )(fp_all)
    return offs[:NB + 1]


def _pool_body(offs_ref,
               idx_ref,
               x_ref,
               wg_ref,
               zn_ref,
               beta_ref,
               xbuf, ibuf, ssum, sem_x, sem_i):
    b = pl.program_id(0)
    start = offs_ref[b]
    end = offs_ref[b + 1]
    n = end - start

    zn_ref[...] = jnp.zeros((SB, D), jnp.float32)
    ssum[...] = jnp.zeros((SB, 1), jnp.float32)

    abase = (start >> 7) << 7
    nch = jax.lax.div(end - abase + CR - 1, CR)
    bank = jax.lax.rem(b, 2) * 4

    def mk_dma(abase_, c, slot):
        o = pl.multiple_of(jnp.minimum(abase_ + c * CR, N - CR), 128)
        return (
            pltpu.make_async_copy(x_ref.at[pl.ds(o, CR), :], xbuf.at[slot],
                                  sem_x.at[slot]),
            pltpu.make_async_copy(idx_ref.at[:, pl.ds(o, CR)], ibuf.at[slot],
                                  sem_i.at[slot]),
        )

    @pl.when(b == 0)
    def _():
        for k in range(3):
            @pl.when(k < nch)
            def _():
                for cp in mk_dma(abase, k, k):
                    cp.start()

    @pl.when(b + 1 < NB)
    def _():
        s2 = offs_ref[b + 1]
        e2 = offs_ref[b + 2]
        ab2 = (s2 >> 7) << 7
        nch2 = jax.lax.div(e2 - ab2 + CR - 1, CR)
        obank = 4 - bank
        for k in range(3):
            @pl.when(k < nch2)
            def _():
                for cp in mk_dma(ab2, k, obank + k):
                    cp.start()

    col = jax.lax.broadcasted_iota(jnp.int32, (1, CR), 1)

    def body(c, c_old):
        slot = bank + jax.lax.rem(c, 4)
        desired = abase + c * CR
        o = jnp.minimum(desired, N - CR)

        @pl.when(c + 3 < nch)
        def _():
            for cp in mk_dma(abase, c + 3, bank + jax.lax.rem(c + 3, 4)):
                cp.start()

        for cp in mk_dma(abase, c, slot):
            cp.wait()

        lo = jnp.maximum(start, desired) - o
        hi = jnp.minimum(end, desired + CR) - o
        xb = xbuf[slot]
        g = jnp.dot(xb, wg_ref[...], preferred_element_type=jnp.float32)
        gr = jnp.transpose(g)
        gr = jnp.where((col >= lo) & (col < hi), gr, _NEG)
        c_new = jnp.maximum(c_old, jnp.max(gr))
        scale = jnp.exp(c_old - c_new)
        e = jnp.exp(gr - c_new)

        l = ibuf[slot] - b * SB
        oh = l == jax.lax.broadcasted_iota(jnp.int32, (SB, 1), 0)
        ohe = jnp.where(oh, e, 0.0).astype(jnp.bfloat16)
        rhs = jnp.concatenate(
            [xb.astype(jnp.bfloat16), jnp.ones((CR, 1), jnp.bfloat16)],
            axis=1)
        acc = jnp.dot(ohe, rhs, preferred_element_type=jnp.float32)
        zn_ref[...] = zn_ref[...] * scale + acc[:, :D]
        ssum[...] = ssum[...] * scale + acc[:, D:]
        return c_new

    jax.lax.fori_loop(0, nch, body, jnp.float32(_NEG))

    s_col = ssum[...]
    beta_ref[...] = s_col / (s_col + 1e-10)
    zn_ref[...] = zn_ref[...] / (s_col + 1e-10)


def _msg_body(zn_ref, beta_ref, wm_ref, bm_ref, out_ref):
    out_ref[...] = (
        jnp.dot(zn_ref[...], wm_ref[...], preferred_element_type=jnp.float32)
        + beta_ref[...] * bm_ref[...])


@functools.partial(jax.jit, static_argnames=("interpret",))
def kernel(x, index, Wg, bg, Wm, bm, interpret=False):
    idx1d = index.astype(jnp.int32)
    idx = idx1d.reshape(1, N)
    if interpret:
        boundaries = jnp.arange(NB + 1, dtype=jnp.int32) * SB
        offs = jnp.searchsorted(idx1d, boundaries, side="left").astype(jnp.int32)
    else:
        offs = _sc_offsets(idx1d)

    grid_spec = pltpu.PrefetchScalarGridSpec(
        num_scalar_prefetch=1,
        grid=(NB,),
        in_specs=[
            pl.BlockSpec(memory_space=pl.ANY),
            pl.BlockSpec(memory_space=pl.ANY),
            pl.BlockSpec((D, 1), lambda b, offs: (0, 0)),
        ],
        out_specs=[
            pl.BlockSpec((SB, D), lambda b, offs: (b, 0)),
            pl.BlockSpec((SB, 1), lambda b, offs: (b, 0)),
        ],
        scratch_shapes=[
            pltpu.VMEM((8, CR, D), jnp.float32),
            pltpu.VMEM((8, 1, CR), jnp.int32),
            pltpu.VMEM((SB, 1), jnp.float32),
            pltpu.SemaphoreType.DMA((8,)),
            pltpu.SemaphoreType.DMA((8,)),
        ],
    )
    zn, beta = pl.pallas_call(
        _pool_body,
        grid_spec=grid_spec,
        out_shape=[
            jax.ShapeDtypeStruct((SPAD, D), jnp.float32),
            jax.ShapeDtypeStruct((SPAD, 1), jnp.float32),
        ],
        interpret=interpret,
    )(offs, idx, x, Wg)

    out = pl.pallas_call(
        _msg_body,
        out_shape=jax.ShapeDtypeStruct((SPAD, D), jnp.float32),
        interpret=interpret,
    )(zn, beta, Wm, bm.reshape(1, D))
    return out[:S]
